# baseline (speedup 1.0000x reference)
import functools

import jax
import jax.numpy as jnp
from jax import lax
from jax.experimental import pallas as pl
from jax.experimental.pallas import tpu as pltpu
from jax.experimental.pallas import tpu_sc as plsc

_K = 8192
_D = 256
_N = 8192

_NB = 16
_TB = _N // _NB
_KB = 8
_CB = _K // _KB

_INT_MAX = 2147483647


def _dist_argmin_body(wsq_ref, z_ref, w_hbm, idx_ref, dsum_ref, w2_ref, sem):
    i = pl.program_id(0)

    @pl.when(i == 0)
    def _load_and_scale_w():
        cp = pltpu.make_async_copy(w_hbm, w2_ref, sem)
        cp.start()
        cp.wait()
        w2_ref[...] = -2.0 * w2_ref[...]

    z = z_ref[...]
    zsq = jnp.sum(z ** 2, axis=1, keepdims=True)
    idsf = lax.broadcasted_iota(jnp.int32, (_TB, _CB), 1).astype(jnp.float32)

    rmin = ridxf = None
    for j in range(_KB):
        w2blk = w2_ref[pl.ds(j * _CB, _CB), :]
        mm2 = lax.dot_general(z, w2blk, (((1,), (1,)), ((), ())),
                              preferred_element_type=jnp.float32)
        d = (zsq + wsq_ref[:, pl.ds(j * _CB, _CB)]) + mm2
        bmin = jnp.min(d, axis=1, keepdims=True)
        bargf = jnp.min(jnp.where(d == bmin, idsf, jnp.inf),
                        axis=1, keepdims=True) + float(j * _CB)
        if rmin is None:
            rmin, ridxf = bmin, bargf
        else:
            upd = bmin < rmin
            rmin = jnp.where(upd, bmin, rmin)
            ridxf = jnp.where(upd, bargf, ridxf)
    idx_ref[...] = ridxf.reshape(_TB).astype(jnp.int32)

    @pl.when(i == 0)
    def _zero():
        dsum_ref[0, 0] = 0.0

    dsum_ref[0, 0] += jnp.sum(rmin)


_dist_argmin = pl.pallas_call(
    _dist_argmin_body,
    grid=(_NB,),
    in_specs=[
        pl.BlockSpec((1, _K), lambda i: (0, 0)),
        pl.BlockSpec((_TB, _D), lambda i: (i, 0)),
        pl.BlockSpec(memory_space=pl.ANY),
    ],
    out_specs=[
        pl.BlockSpec((_TB,), lambda i: (i,)),
        pl.BlockSpec((1, 1), lambda i: (0, 0), memory_space=pltpu.SMEM),
    ],
    out_shape=[
        jax.ShapeDtypeStruct((_N,), jnp.int32),
        jax.ShapeDtypeStruct((1, 1), jnp.float32),
    ],
    scratch_shapes=[pltpu.VMEM((_K, _D), jnp.float32),
                    pltpu.SemaphoreType.DMA],
)


_NC = 2
_NS = 16
_NW = _NC * _NS
_RPW = _N // _NW


@functools.lru_cache(maxsize=None)
def _make_sc_gather():
    mesh = plsc.VectorSubcoreMesh(core_axis_name="c", subcore_axis_name="s")

    @functools.partial(
        pl.kernel,
        mesh=mesh,
        out_type=jax.ShapeDtypeStruct((_N, _D), jnp.float32),
        scratch_types=[
            pltpu.VMEM((_RPW,), jnp.int32),
            pltpu.VMEM((_RPW, _D), jnp.float32),
            pltpu.SemaphoreType.DMA,
        ],
    )
    def _sc_gather(w_hbm, idx_hbm, out_hbm, idx_v, rows_v, sem):
        wid = lax.axis_index("s") * _NC + lax.axis_index("c")
        base = wid * _RPW
        pltpu.sync_copy(idx_hbm.at[pl.ds(base, _RPW)], idx_v)
        pltpu.async_copy(w_hbm.at[idx_v], rows_v, sem).wait()
        pltpu.sync_copy(rows_v, out_hbm.at[pl.ds(base, _RPW)])

    return _sc_gather


def kernel(z, W):
    zp = jnp.transpose(z, (0, 2, 3, 1))
    z_flat = zp.reshape(-1, _D)
    wsq = jnp.sum(W ** 2, axis=1)
    idx, dsum = _dist_argmin(wsq.reshape(1, _K), z_flat, W)
    zq_flat = _make_sc_gather()(W, idx)
    m = dsum[0, 0] / jnp.float32(_N * _D)
    vq_loss = 0.25 * m + m
    out = jnp.transpose(zq_flat.reshape(8, 32, 32, _D), (0, 3, 1, 2))
    return (out, vq_loss, idx)


# speedup vs baseline: 1.0672x; 1.0672x over previous
import functools

import jax
import jax.numpy as jnp
from jax import lax
from jax.experimental import pallas as pl
from jax.experimental.pallas import tpu as pltpu
from jax.experimental.pallas import tpu_sc as plsc

_K = 8192
_D = 256
_N = 8192

_NB = 8
_TB = _N // _NB
_KB = 8
_CB = _K // _KB

_INT_MAX = 2147483647


def _dist_argmin_body(wsq_ref, z_ref, w_hbm, idx_ref, dsum_ref, w2_ref, sem):
    i = pl.program_id(0)

    @pl.when(i == 0)
    def _load_and_scale_w():
        cp = pltpu.make_async_copy(w_hbm, w2_ref, sem)
        cp.start()
        cp.wait()
        w2_ref[...] = -2.0 * w2_ref[...]

    z = z_ref[...]
    zsq = jnp.sum(z ** 2, axis=1, keepdims=True)
    idsf = lax.broadcasted_iota(jnp.int32, (_TB, _CB), 1).astype(jnp.float32)

    rmin = ridxf = None
    for j in range(_KB):
        w2blk = w2_ref[pl.ds(j * _CB, _CB), :]
        mm2 = lax.dot_general(z, w2blk, (((1,), (1,)), ((), ())),
                              preferred_element_type=jnp.float32)
        d = (zsq + wsq_ref[:, pl.ds(j * _CB, _CB)]) + mm2
        bmin = jnp.min(d, axis=1, keepdims=True)
        bargf = jnp.min(jnp.where(d == bmin, idsf, jnp.inf),
                        axis=1, keepdims=True) + float(j * _CB)
        if rmin is None:
            rmin, ridxf = bmin, bargf
        else:
            upd = bmin < rmin
            rmin = jnp.where(upd, bmin, rmin)
            ridxf = jnp.where(upd, bargf, ridxf)
    idx_ref[...] = ridxf.astype(jnp.int32)

    @pl.when(i == 0)
    def _zero():
        dsum_ref[0, 0] = 0.0

    dsum_ref[0, 0] += jnp.sum(rmin)


_dist_argmin = pl.pallas_call(
    _dist_argmin_body,
    grid=(_NB,),
    in_specs=[
        pl.BlockSpec((1, _K), lambda i: (0, 0)),
        pl.BlockSpec((_TB, _D), lambda i: (i, 0)),
        pl.BlockSpec(memory_space=pl.ANY),
    ],
    out_specs=[
        pl.BlockSpec((_TB, 1), lambda i: (i, 0)),
        pl.BlockSpec((1, 1), lambda i: (0, 0), memory_space=pltpu.SMEM),
    ],
    out_shape=[
        jax.ShapeDtypeStruct((_N, 1), jnp.int32),
        jax.ShapeDtypeStruct((1, 1), jnp.float32),
    ],
    scratch_shapes=[pltpu.VMEM((_K, _D), jnp.float32),
                    pltpu.SemaphoreType.DMA],
)


_NC = 2
_NS = 16
_NW = _NC * _NS
_RPW = _N // _NW


@functools.lru_cache(maxsize=None)
def _make_sc_gather():
    mesh = plsc.VectorSubcoreMesh(core_axis_name="c", subcore_axis_name="s")

    @functools.partial(
        pl.kernel,
        mesh=mesh,
        out_type=jax.ShapeDtypeStruct((_N, _D), jnp.float32),
        scratch_types=[
            pltpu.VMEM((_RPW,), jnp.int32),
            pltpu.VMEM((_RPW, _D), jnp.float32),
            pltpu.SemaphoreType.DMA,
        ],
    )
    def _sc_gather(w_hbm, idx_hbm, out_hbm, idx_v, rows_v, sem):
        wid = lax.axis_index("s") * _NC + lax.axis_index("c")
        base = wid * _RPW
        pltpu.sync_copy(idx_hbm.at[pl.ds(base, _RPW)], idx_v)
        pltpu.async_copy(w_hbm.at[idx_v], rows_v, sem).wait()
        pltpu.sync_copy(rows_v, out_hbm.at[pl.ds(base, _RPW)])

    return _sc_gather


def kernel(z, W):
    zp = jnp.transpose(z, (0, 2, 3, 1))
    z_flat = zp.reshape(-1, _D)
    wsq = jnp.sum(W ** 2, axis=1)
    idx2, dsum = _dist_argmin(wsq.reshape(1, _K), z_flat, W)
    idx = idx2.reshape(_N)
    zq_flat = _make_sc_gather()(W, idx)
    m = dsum[0, 0] / jnp.float32(_N * _D)
    vq_loss = 0.25 * m + m
    out = jnp.transpose(zq_flat.reshape(8, 32, 32, _D), (0, 3, 1, 2))
    return (out, vq_loss, idx)
